# baseline (speedup 1.0000x reference)
import functools

import jax
import jax.numpy as jnp
from jax import lax
from jax.experimental import pallas as pl
from jax.experimental.pallas import tpu as pltpu
from jax.experimental.pallas import tpu_sc as plsc

_NUM_CORES = 2
_NUM_SUBCORES = 16
_NW = _NUM_CORES * _NUM_SUBCORES
_LANES = 16

_NB = 8


def _transform_table(data_table, W, b2, unk, blk):
    V, D = data_table.shape
    O = W.shape[0]
    grid = V // blk

    def body(x_ref, w_ref, b_ref, unk_ref, out_ref):
        t = lax.dot_general(
            x_ref[...], w_ref[...],
            (((1,), (1,)), ((), ())),
            preferred_element_type=jnp.float32,
        )
        out_ref[...] = t + b_ref[...]

        @pl.when(pl.program_id(0) == grid - 1)
        def _():
            out_ref[blk - 1, :] = unk_ref[0, :]

    return pl.pallas_call(
        body,
        grid=(grid,),
        in_specs=[
            pl.BlockSpec((blk, D), lambda i: (i, 0)),
            pl.BlockSpec((O, D), lambda i: (0, 0)),
            pl.BlockSpec((1, O), lambda i: (0, 0)),
            pl.BlockSpec((1, O), lambda i: (0, 0)),
        ],
        out_specs=pl.BlockSpec((blk, O), lambda i: (i, 0)),
        out_shape=jax.ShapeDtypeStruct((V, O), jnp.float32),
    )(data_table, W, b2, unk)


@functools.lru_cache(maxsize=None)
def _make_gather(V, O, B, H):
    per_w = B // _NW
    n_chunks = per_w // _NB
    mesh = plsc.VectorSubcoreMesh(core_axis_name="c", subcore_axis_name="s")

    cols = list(range(0, H - _LANES, _LANES)) + [H - _LANES]
    HP = (H + 7) // 8 * 8

    @functools.partial(
        pl.kernel,
        mesh=mesh,
        compiler_params=pltpu.CompilerParams(use_tc_tiling_on_sc=False),
        out_type=jax.ShapeDtypeStruct((B, H, O), jnp.float32),
        scratch_types=[
            pltpu.VMEM((_NB * H,), jnp.int32),
            pltpu.VMEM((_NB, 64), jnp.int32),
            pltpu.VMEM((_NB * HP, O), jnp.float32),
            pltpu.SemaphoreType.DMA,
            pltpu.SemaphoreType.DMA,
        ],
    )
    def gather_kernel(tbl_hbm, idx_hbm, out_hbm, idxv, idx_map, rows,
                      sem, wsem):
        wid = lax.axis_index("s") * _NUM_CORES + lax.axis_index("c")

        def chunk_body(ci, carry):
            b0 = wid * per_w + ci * _NB
            pltpu.sync_copy(idx_hbm.at[pl.ds(b0 * H, _NB * H)], idxv)
            for bb in range(_NB):
                fill = None
                for c in cols:
                    v = idxv[pl.ds(bb * H + c, _LANES)]
                    m = jnp.where(v < 1, V - 1, v - 1)
                    if c == cols[-2]:
                        fill = m
                    if c == cols[-1] and HP > H:
                        idx_map[bb, pl.ds(HP - _LANES, _LANES)] = fill
                    idx_map[bb, pl.ds(c, _LANES)] = m
            copies = [
                pltpu.async_copy(
                    tbl_hbm.at[idx_map.at[bb, pl.ds(0, HP)]],
                    rows.at[pl.ds(bb * HP, HP)],
                    sem,
                )
                for bb in range(_NB)
            ]
            for c in copies:
                c.wait()
            wcopies = [
                pltpu.async_copy(
                    rows.at[pl.ds(bb * HP, H)],
                    out_hbm.at[b0 + bb],
                    wsem,
                )
                for bb in range(_NB)
            ]
            for c in wcopies:
                c.wait()
            return carry

        lax.fori_loop(0, n_chunks, chunk_body, 0)

    return gather_kernel


def kernel(indices, data_table, unknown_table, W, b):
    V, D = data_table.shape
    O = W.shape[0]
    B, H = indices.shape

    T = _transform_table(
        data_table, W, b.reshape(1, O), unknown_table, blk=5000
    )
    idx_flat = indices.reshape(B * H).astype(jnp.int32)
    return _make_gather(V, O, B, H)(T, idx_flat)
